# Initial kernel scaffold; baseline (speedup 1.0000x reference)
import jax
import jax.numpy as jnp
from jax.experimental import pallas as pl


def kernel(x, edge_index, edge_attr, W1, b1, W2, b2):
    raise NotImplementedError("write your pallas kernel here")


# speedup vs baseline: 9.9774x; 9.9774x over previous
import functools

import jax
import jax.numpy as jnp
from jax import lax
from jax.experimental import pallas as pl
from jax.experimental.pallas import tpu as pltpu
import jax.experimental.pallas.tpu_sc as plsc

NC = 2
NS = 16
NW = NC * NS


def _sc_mesh():
    return plsc.VectorSubcoreMesh(
        core_axis_name="c", subcore_axis_name="s", num_cores=NC, num_subcores=NS
    )


def _degree_pass(colr, e1r, e2r, zslab, np_):
    KD, CD = colr.shape[1], colr.shape[2]
    slab = np_ // NS

    @functools.partial(
        pl.kernel,
        out_type=jax.ShapeDtypeStruct((NC, 2, np_), jnp.float32),
        mesh=_sc_mesh(),
        scratch_types=[
            pltpu.VMEM((KD, CD), jnp.int32),
            pltpu.VMEM((KD, CD), jnp.float32),
            pltpu.VMEM((KD, CD), jnp.float32),
            pltpu.VMEM_SHARED((np_,), jnp.float32),
            pltpu.VMEM_SHARED((np_,), jnp.float32),
            pltpu.SemaphoreType.DMA,
        ],
    )
    def deg_kernel(col_hbm, e1_hbm, e2_hbm, z_hbm, out_hbm, colv, e1v, e2v, acc1, acc2, sem):
        c = lax.axis_index("c")
        s = lax.axis_index("s")
        w = c * NS + s
        pltpu.sync_copy(col_hbm.at[w], colv)
        pltpu.sync_copy(e1_hbm.at[w], e1v)
        pltpu.sync_copy(e2_hbm.at[w], e2v)
        pltpu.sync_copy(z_hbm, acc1.at[pl.ds(s * slab, slab)])
        pltpu.sync_copy(z_hbm, acc2.at[pl.ds(s * slab, slab)])
        plsc.subcore_barrier()
        cps = []
        for k in range(KD):
            cps.append(pltpu.async_copy(e1v.at[k], acc1.at[colv.at[k]], sem, add=True))
            cps.append(pltpu.async_copy(e2v.at[k], acc2.at[colv.at[k]], sem, add=True))
        for cp in cps:
            cp.wait()
        plsc.subcore_barrier()
        pltpu.sync_copy(acc1.at[pl.ds(s * slab, slab)], out_hbm.at[c, 0, pl.ds(s * slab, slab)])
        pltpu.sync_copy(acc2.at[pl.ds(s * slab, slab)], out_hbm.at[c, 1, pl.ds(s * slab, slab)])

    return deg_kernel(colr, e1r, e2r, zslab)


def _spmm_pass(y2h, rowr, colr, ewr, zslab, np_, dh):
    KE, CE = rowr.shape[1], rowr.shape[2]
    slab = np_ // NS

    @functools.partial(
        pl.kernel,
        out_type=jax.ShapeDtypeStruct((NC, np_, dh), jnp.float32),
        mesh=_sc_mesh(),
        scratch_types=[
            pltpu.VMEM((KE, CE), jnp.int32),
            pltpu.VMEM((KE, CE), jnp.int32),
            pltpu.VMEM((KE, CE), jnp.float32),
            pltpu.VMEM((CE, dh), jnp.float32),
            pltpu.VMEM_SHARED((np_, dh), jnp.float32),
            pltpu.SemaphoreType.DMA,
            pltpu.SemaphoreType.DMA,
        ],
        compiler_params=pltpu.CompilerParams(use_tc_tiling_on_sc=False),
    )
    def spmm_kernel(y_hbm, row_hbm, col_hbm, ew_hbm, z_hbm, out_hbm,
                    rowv, colv, ewv, buf, acc, gsem, ssem):
        c = lax.axis_index("c")
        s = lax.axis_index("s")
        pltpu.sync_copy(row_hbm.at[s], rowv)
        pltpu.sync_copy(col_hbm.at[s], colv)
        pltpu.sync_copy(ew_hbm.at[s], ewv)
        pltpu.sync_copy(z_hbm, acc.at[pl.ds(s * slab, slab)])
        plsc.subcore_barrier()

        def chunk(k, carry):
            pltpu.async_copy(y_hbm.at[c].at[rowv.at[k]], buf, gsem).wait()

            def scale_group(g, carry2):
                v = ewv[k, pl.ds(g * 16, 16)]
                for i2 in range(16):
                    sc = v[i2]
                    r = g * 16 + i2
                    for j in range(dh // 16):
                        buf[r, pl.ds(j * 16, 16)] = buf[r, pl.ds(j * 16, 16)] * sc
                return carry2

            lax.fori_loop(0, CE // 16, scale_group, 0)
            pltpu.async_copy(buf, acc.at[colv.at[k]], ssem, add=True).wait()
            return carry

        lax.fori_loop(0, KE, chunk, 0)
        plsc.subcore_barrier()
        pltpu.sync_copy(acc.at[pl.ds(s * slab, slab)], out_hbm.at[c, pl.ds(s * slab, slab)])

    return spmm_kernel(y2h, rowr, colr, ewr, zslab)


def _dinv_pass(degp4):
    def body(dp_ref, o_ref):
        deg = dp_ref[0] + dp_ref[1] + 1.0
        o_ref[...] = jnp.where(deg > 0, lax.rsqrt(jnp.maximum(deg, 1e-12)), 0.0)

    return pl.pallas_call(
        body, out_shape=jax.ShapeDtypeStruct(degp4.shape[1:], jnp.float32)
    )(degp4)


_BLK = 1280


def _mm_scale_pass(x, W, dinvb):
    n, _ = x.shape
    d = W.shape[1]
    dh = d // 2
    g = n // _BLK

    def body(x_ref, w_ref, dv_ref, o_ref):
        xw = jnp.dot(x_ref[...], w_ref[...], preferred_element_type=jnp.float32)
        t = dv_ref[...] * xw
        o_ref[0] = t[:, :dh]
        o_ref[1] = t[:, dh:]

    return pl.pallas_call(
        body,
        grid=(g,),
        in_specs=[
            pl.BlockSpec((_BLK, x.shape[1]), lambda i: (i, 0)),
            pl.BlockSpec(W.shape, lambda i: (0, 0)),
            pl.BlockSpec((_BLK, d), lambda i: (i, 0)),
        ],
        out_specs=pl.BlockSpec((2, _BLK, dh), lambda i: (0, i, 0)),
        out_shape=jax.ShapeDtypeStruct((2, n, dh), jnp.float32),
    )(x, W, dinvb)


def _mid_pass(accp, y1, dinv1b, b1, W2, dinv2b):
    _, n, dh = y1.shape
    d = 2 * dh
    d2 = W2.shape[1]
    d2h = d2 // 2
    g = n // _BLK

    def body(a_ref, y_ref, d1_ref, b_ref, w_ref, d2_ref, o_ref):
        acc = jnp.concatenate([a_ref[0], a_ref[1]], axis=1)
        y = jnp.concatenate([y_ref[0], y_ref[1]], axis=1)
        t = d1_ref[...] * (acc + y) + b_ref[...]
        h = jnp.maximum(t, 0.0)
        hw = jnp.dot(h, w_ref[...], preferred_element_type=jnp.float32)
        t2 = d2_ref[...] * hw
        o_ref[0] = t2[:, :d2h]
        o_ref[1] = t2[:, d2h:]

    return pl.pallas_call(
        body,
        grid=(g,),
        in_specs=[
            pl.BlockSpec((2, _BLK, dh), lambda i: (0, i, 0)),
            pl.BlockSpec((2, _BLK, dh), lambda i: (0, i, 0)),
            pl.BlockSpec((_BLK, d), lambda i: (i, 0)),
            pl.BlockSpec(b1.shape, lambda i: (0, 0)),
            pl.BlockSpec(W2.shape, lambda i: (0, 0)),
            pl.BlockSpec((_BLK, d2), lambda i: (i, 0)),
        ],
        out_specs=pl.BlockSpec((2, _BLK, d2h), lambda i: (0, i, 0)),
        out_shape=jax.ShapeDtypeStruct((2, n, d2h), jnp.float32),
    )(accp, y1, dinv1b, b1, W2, dinv2b)


def _final_pass(accp, y2, dinv2b, b2):
    _, n, dh = y2.shape
    d = 2 * dh
    g = n // _BLK

    def body(a_ref, y_ref, dv_ref, b_ref, o_ref):
        acc = jnp.concatenate([a_ref[0], a_ref[1]], axis=1)
        y = jnp.concatenate([y_ref[0], y_ref[1]], axis=1)
        o_ref[...] = dv_ref[...] * (acc + y) + b_ref[...]

    return pl.pallas_call(
        body,
        grid=(g,),
        in_specs=[
            pl.BlockSpec((2, _BLK, dh), lambda i: (0, i, 0)),
            pl.BlockSpec((2, _BLK, dh), lambda i: (0, i, 0)),
            pl.BlockSpec((_BLK, d), lambda i: (i, 0)),
            pl.BlockSpec(b2.shape, lambda i: (0, 0)),
        ],
        out_specs=pl.BlockSpec((_BLK, d), lambda i: (i, 0)),
        out_shape=jax.ShapeDtypeStruct((n, d), jnp.float32),
    )(accp, y2, dinv2b, b2)


def kernel(x, edge_index, edge_attr, W1, b1, W2, b2):
    n, d_in = x.shape
    e = edge_index.shape[1]
    d_hid = W1.shape[1]
    d_out = W2.shape[1]
    np_ = ((n + _BLK - 1) // _BLK) * _BLK

    row = edge_index[0]
    col = edge_index[1]
    ew1 = edge_attr[:, 0]
    ew2 = edge_attr[:, 1]

    CE = 80
    KE = e // (NS * CE)
    rowr = row.reshape(NS, KE, CE)
    colr = col.reshape(NS, KE, CE)
    ew1r = ew1.reshape(NS, KE, CE)
    ew2r = ew2.reshape(NS, KE, CE)
    CD = 125
    KD = e // (NW * CD)
    colrd = col.reshape(NW, KD, CD)
    e1rd = ew1.reshape(NW, KD, CD)
    e2rd = ew2.reshape(NW, KD, CD)

    dh = d_hid // 2
    zdeg = jnp.zeros((np_ // NS,), jnp.float32)
    zacc = jnp.zeros((np_ // NS, dh), jnp.float32)
    x_p = jnp.pad(x, ((0, np_ - n), (0, 0)))

    degp = _degree_pass(colrd, e1rd, e2rd, zdeg, np_)
    dinvp = _dinv_pass(degp.reshape(NC, 2, np_ // 128, 128))
    dinv1b = jnp.broadcast_to(dinvp[0].reshape(np_, 1), (np_, d_hid))
    dinv2b = jnp.broadcast_to(dinvp[1].reshape(np_, 1), (np_, d_out))

    y1 = _mm_scale_pass(x_p, W1, dinv1b)
    acc1 = _spmm_pass(y1, rowr, colr, ew1r, zacc, np_, dh)
    y2 = _mid_pass(acc1, y1, dinv1b, b1.reshape(1, -1), W2, dinv2b)
    acc2 = _spmm_pass(y2, rowr, colr, ew2r, zacc, np_, d_out // 2)
    outp = _final_pass(acc2, y2, dinv2b, b2.reshape(1, -1))
    return outp[:n]


# speedup vs baseline: 16.2643x; 1.6301x over previous
import functools

import jax
import jax.numpy as jnp
from jax import lax
from jax.experimental import pallas as pl
from jax.experimental.pallas import tpu as pltpu
import jax.experimental.pallas.tpu_sc as plsc

NC = 2
NS = 16
NW = NC * NS


def _sc_mesh():
    return plsc.VectorSubcoreMesh(
        core_axis_name="c", subcore_axis_name="s", num_cores=NC, num_subcores=NS
    )


def _degree_pass(colr, e1r, e2r, zslab, np_):
    KD, CD = colr.shape[1], colr.shape[2]
    slab = np_ // NS

    @functools.partial(
        pl.kernel,
        out_type=jax.ShapeDtypeStruct((NC, 2, np_), jnp.float32),
        mesh=_sc_mesh(),
        scratch_types=[
            pltpu.VMEM((KD, CD), jnp.int32),
            pltpu.VMEM((KD, CD), jnp.float32),
            pltpu.VMEM((KD, CD), jnp.float32),
            pltpu.VMEM_SHARED((np_,), jnp.float32),
            pltpu.VMEM_SHARED((np_,), jnp.float32),
            pltpu.SemaphoreType.DMA,
        ],
    )
    def deg_kernel(col_hbm, e1_hbm, e2_hbm, z_hbm, out_hbm, colv, e1v, e2v, acc1, acc2, sem):
        c = lax.axis_index("c")
        s = lax.axis_index("s")
        w = c * NS + s
        pltpu.sync_copy(col_hbm.at[w], colv)
        pltpu.sync_copy(e1_hbm.at[w], e1v)
        pltpu.sync_copy(e2_hbm.at[w], e2v)
        pltpu.sync_copy(z_hbm, acc1.at[pl.ds(s * slab, slab)])
        pltpu.sync_copy(z_hbm, acc2.at[pl.ds(s * slab, slab)])
        plsc.subcore_barrier()
        cps = []
        for k in range(KD):
            cps.append(pltpu.async_copy(e1v.at[k], acc1.at[colv.at[k]], sem, add=True))
            cps.append(pltpu.async_copy(e2v.at[k], acc2.at[colv.at[k]], sem, add=True))
        for cp in cps:
            cp.wait()
        plsc.subcore_barrier()
        pltpu.sync_copy(acc1.at[pl.ds(s * slab, slab)], out_hbm.at[c, 0, pl.ds(s * slab, slab)])
        pltpu.sync_copy(acc2.at[pl.ds(s * slab, slab)], out_hbm.at[c, 1, pl.ds(s * slab, slab)])

    return deg_kernel(colr, e1r, e2r, zslab)


_NB = 5


def _spmm_pass(y2h, rowr, colr, ewr, zslab, np_, dh):
    KE, CE = rowr.shape[1], rowr.shape[2]
    slab = np_ // NS

    @functools.partial(
        pl.kernel,
        out_type=jax.ShapeDtypeStruct((NC, np_, dh), jnp.float32),
        mesh=_sc_mesh(),
        scratch_types=[
            pltpu.VMEM((KE, CE), jnp.int32),
            pltpu.VMEM((KE, CE), jnp.int32),
            pltpu.VMEM((KE, CE), jnp.float32),
            [pltpu.VMEM((CE, dh), jnp.float32)] * _NB,
            pltpu.VMEM_SHARED((np_, dh), jnp.float32),
            [pltpu.SemaphoreType.DMA] * _NB,
            [pltpu.SemaphoreType.DMA] * _NB,
        ],
        compiler_params=pltpu.CompilerParams(use_tc_tiling_on_sc=False),
    )
    def spmm_kernel(y_hbm, row_hbm, col_hbm, ew_hbm, z_hbm, out_hbm,
                    rowv, colv, ewv, bufs, acc, gsems, ssems):
        c = lax.axis_index("c")
        s = lax.axis_index("s")
        pltpu.sync_copy(row_hbm.at[s], rowv)
        pltpu.sync_copy(col_hbm.at[s], colv)
        pltpu.sync_copy(ew_hbm.at[s], ewv)
        pltpu.sync_copy(z_hbm, acc.at[pl.ds(s * slab, slab)])
        plsc.subcore_barrier()

        def gather_start(k, b):
            pltpu.async_copy(y_hbm.at[c].at[rowv.at[k]], bufs[b], gsems[b])

        def gather_wait(k, b):
            pltpu.make_async_copy(y_hbm.at[c].at[rowv.at[k]], bufs[b], gsems[b]).wait()

        def scatter_start(k, b):
            pltpu.async_copy(bufs[b], acc.at[colv.at[k]], ssems[b], add=True)

        def scatter_wait(k, b):
            pltpu.make_async_copy(bufs[b], acc.at[colv.at[k]], ssems[b]).wait()

        def scale(k, b):
            buf = bufs[b]

            def scale_group(g, carry2):
                v = ewv[k, pl.ds(g * 16, 16)]
                for i2 in range(16):
                    sc = v[i2]
                    r = g * 16 + i2
                    for j in range(dh // 16):
                        buf[r, pl.ds(j * 16, 16)] = buf[r, pl.ds(j * 16, 16)] * sc
                return carry2

            lax.fori_loop(0, CE // 16, scale_group, 0)

        gather_start(0, 0)
        gather_start(1, 1)
        for b in range(_NB):
            if b + 2 < _NB:
                gather_start(b + 2, b + 2)
            else:
                scatter_wait(b + 2 - _NB, (b + 2) % _NB)
                gather_start(b + 2, (b + 2) % _NB)
            gather_wait(b, b)
            scale(b, b)
            scatter_start(b, b)

        def round_body(r, carry):
            for b in range(_NB):
                k = r * _NB + b
                b2 = (b + 2) % _NB
                scatter_wait(k - 3, b2)
                gather_start(k + 2, b2)
                gather_wait(k, b)
                scale(k, b)
                scatter_start(k, b)
            return carry

        lax.fori_loop(1, KE // _NB - 1, round_body, 0)

        for b in range(_NB):
            k = KE - _NB + b
            b2 = (b + 2) % _NB
            scatter_wait(k - 3, b2)
            if k + 2 < KE:
                gather_start(k + 2, b2)
            gather_wait(k, b)
            scale(k, b)
            scatter_start(k, b)
        for k in (KE - 3, KE - 2, KE - 1):
            scatter_wait(k, k % _NB)

        plsc.subcore_barrier()
        pltpu.sync_copy(acc.at[pl.ds(s * slab, slab)], out_hbm.at[c, pl.ds(s * slab, slab)])

    return spmm_kernel(y2h, rowr, colr, ewr, zslab)


def _dinv_pass(degp4):
    def body(dp_ref, o_ref):
        deg = dp_ref[0] + dp_ref[1] + 1.0
        o_ref[...] = jnp.where(deg > 0, lax.rsqrt(jnp.maximum(deg, 1e-12)), 0.0)

    return pl.pallas_call(
        body, out_shape=jax.ShapeDtypeStruct(degp4.shape[1:], jnp.float32)
    )(degp4)


_BLK = 1280


def _mm_scale_pass(x, W, dinvb):
    n, _ = x.shape
    d = W.shape[1]
    dh = d // 2
    g = n // _BLK

    def body(x_ref, w_ref, dv_ref, o_ref):
        xw = jnp.dot(x_ref[...], w_ref[...], preferred_element_type=jnp.float32)
        t = dv_ref[...] * xw
        o_ref[0] = t[:, :dh]
        o_ref[1] = t[:, dh:]

    return pl.pallas_call(
        body,
        grid=(g,),
        in_specs=[
            pl.BlockSpec((_BLK, x.shape[1]), lambda i: (i, 0)),
            pl.BlockSpec(W.shape, lambda i: (0, 0)),
            pl.BlockSpec((_BLK, d), lambda i: (i, 0)),
        ],
        out_specs=pl.BlockSpec((2, _BLK, dh), lambda i: (0, i, 0)),
        out_shape=jax.ShapeDtypeStruct((2, n, dh), jnp.float32),
    )(x, W, dinvb)


def _mid_pass(accp, y1, dinv1b, b1, W2, dinv2b):
    _, n, dh = y1.shape
    d = 2 * dh
    d2 = W2.shape[1]
    d2h = d2 // 2
    g = n // _BLK

    def body(a_ref, y_ref, d1_ref, b_ref, w_ref, d2_ref, o_ref):
        acc = jnp.concatenate([a_ref[0], a_ref[1]], axis=1)
        y = jnp.concatenate([y_ref[0], y_ref[1]], axis=1)
        t = d1_ref[...] * (acc + y) + b_ref[...]
        h = jnp.maximum(t, 0.0)
        hw = jnp.dot(h, w_ref[...], preferred_element_type=jnp.float32)
        t2 = d2_ref[...] * hw
        o_ref[0] = t2[:, :d2h]
        o_ref[1] = t2[:, d2h:]

    return pl.pallas_call(
        body,
        grid=(g,),
        in_specs=[
            pl.BlockSpec((2, _BLK, dh), lambda i: (0, i, 0)),
            pl.BlockSpec((2, _BLK, dh), lambda i: (0, i, 0)),
            pl.BlockSpec((_BLK, d), lambda i: (i, 0)),
            pl.BlockSpec(b1.shape, lambda i: (0, 0)),
            pl.BlockSpec(W2.shape, lambda i: (0, 0)),
            pl.BlockSpec((_BLK, d2), lambda i: (i, 0)),
        ],
        out_specs=pl.BlockSpec((2, _BLK, d2h), lambda i: (0, i, 0)),
        out_shape=jax.ShapeDtypeStruct((2, n, d2h), jnp.float32),
    )(accp, y1, dinv1b, b1, W2, dinv2b)


def _final_pass(accp, y2, dinv2b, b2):
    _, n, dh = y2.shape
    d = 2 * dh
    g = n // _BLK

    def body(a_ref, y_ref, dv_ref, b_ref, o_ref):
        acc = jnp.concatenate([a_ref[0], a_ref[1]], axis=1)
        y = jnp.concatenate([y_ref[0], y_ref[1]], axis=1)
        o_ref[...] = dv_ref[...] * (acc + y) + b_ref[...]

    return pl.pallas_call(
        body,
        grid=(g,),
        in_specs=[
            pl.BlockSpec((2, _BLK, dh), lambda i: (0, i, 0)),
            pl.BlockSpec((2, _BLK, dh), lambda i: (0, i, 0)),
            pl.BlockSpec((_BLK, d), lambda i: (i, 0)),
            pl.BlockSpec(b2.shape, lambda i: (0, 0)),
        ],
        out_specs=pl.BlockSpec((_BLK, d), lambda i: (i, 0)),
        out_shape=jax.ShapeDtypeStruct((n, d), jnp.float32),
    )(accp, y2, dinv2b, b2)


def kernel(x, edge_index, edge_attr, W1, b1, W2, b2):
    n, d_in = x.shape
    e = edge_index.shape[1]
    d_hid = W1.shape[1]
    d_out = W2.shape[1]
    np_ = ((n + _BLK - 1) // _BLK) * _BLK

    row = edge_index[0]
    col = edge_index[1]
    ew1 = edge_attr[:, 0]
    ew2 = edge_attr[:, 1]

    CE = 80
    KE = e // (NS * CE)
    rowr = row.reshape(NS, KE, CE)
    colr = col.reshape(NS, KE, CE)
    ew1r = ew1.reshape(NS, KE, CE)
    ew2r = ew2.reshape(NS, KE, CE)
    CD = 125
    KD = e // (NW * CD)
    colrd = col.reshape(NW, KD, CD)
    e1rd = ew1.reshape(NW, KD, CD)
    e2rd = ew2.reshape(NW, KD, CD)

    dh = d_hid // 2
    zdeg = jnp.zeros((np_ // NS,), jnp.float32)
    zacc = jnp.zeros((np_ // NS, dh), jnp.float32)
    x_p = jnp.pad(x, ((0, np_ - n), (0, 0)))

    degp = _degree_pass(colrd, e1rd, e2rd, zdeg, np_)
    dinvp = _dinv_pass(degp.reshape(NC, 2, np_ // 128, 128))
    dinv1b = jnp.broadcast_to(dinvp[0].reshape(np_, 1), (np_, d_hid))
    dinv2b = jnp.broadcast_to(dinvp[1].reshape(np_, 1), (np_, d_out))

    y1 = _mm_scale_pass(x_p, W1, dinv1b)
    acc1 = _spmm_pass(y1, rowr, colr, ew1r, zacc, np_, dh)
    y2 = _mid_pass(acc1, y1, dinv1b, b1.reshape(1, -1), W2, dinv2b)
    acc2 = _spmm_pass(y2, rowr, colr, ew2r, zacc, np_, d_out // 2)
    outp = _final_pass(acc2, y2, dinv2b, b2.reshape(1, -1))
    return outp[:n]


# speedup vs baseline: 31.4332x; 1.9326x over previous
import functools

import jax
import jax.numpy as jnp
from jax import lax
from jax.experimental import pallas as pl
from jax.experimental.pallas import tpu as pltpu
import jax.experimental.pallas.tpu_sc as plsc

NC = 2
NS = 16
NW = NC * NS


def _sc_mesh():
    return plsc.VectorSubcoreMesh(
        core_axis_name="c", subcore_axis_name="s", num_cores=NC, num_subcores=NS
    )


def _degree_pass(colr, e1r, e2r, zslab, np_):
    KD, CD = colr.shape[1], colr.shape[2]
    slab = np_ // NS

    @functools.partial(
        pl.kernel,
        out_type=jax.ShapeDtypeStruct((NC, 2, np_), jnp.float32),
        mesh=_sc_mesh(),
        scratch_types=[
            pltpu.VMEM((KD, CD), jnp.int32),
            pltpu.VMEM((KD, CD), jnp.float32),
            pltpu.VMEM((KD, CD), jnp.float32),
            pltpu.VMEM_SHARED((np_,), jnp.float32),
            pltpu.VMEM_SHARED((np_,), jnp.float32),
            pltpu.SemaphoreType.DMA,
        ],
    )
    def deg_kernel(col_hbm, e1_hbm, e2_hbm, z_hbm, out_hbm, colv, e1v, e2v, acc1, acc2, sem):
        c = lax.axis_index("c")
        s = lax.axis_index("s")
        w = c * NS + s
        pltpu.sync_copy(col_hbm.at[w], colv)
        pltpu.sync_copy(e1_hbm.at[w], e1v)
        pltpu.sync_copy(e2_hbm.at[w], e2v)
        pltpu.sync_copy(z_hbm, acc1.at[pl.ds(s * slab, slab)])
        pltpu.sync_copy(z_hbm, acc2.at[pl.ds(s * slab, slab)])
        plsc.subcore_barrier()
        cps = []
        for k in range(KD):
            cps.append(pltpu.async_copy(e1v.at[k], acc1.at[colv.at[k]], sem, add=True))
            cps.append(pltpu.async_copy(e2v.at[k], acc2.at[colv.at[k]], sem, add=True))
        for cp in cps:
            cp.wait()
        plsc.subcore_barrier()
        pltpu.sync_copy(acc1.at[pl.ds(s * slab, slab)], out_hbm.at[c, 0, pl.ds(s * slab, slab)])
        pltpu.sync_copy(acc2.at[pl.ds(s * slab, slab)], out_hbm.at[c, 1, pl.ds(s * slab, slab)])

    return deg_kernel(colr, e1r, e2r, zslab)


_NB = 5


def _spmm_pass(y2h, rcr, ewr, zslab, np_, dh, KE_CE):
    KE = KE_CE[0]
    CE = KE_CE[1]
    slab = np_ // NS

    @functools.partial(
        pl.kernel,
        out_type=jax.ShapeDtypeStruct((NC, np_, dh), jnp.float32),
        mesh=_sc_mesh(),
        scratch_types=[
            pltpu.VMEM((_NB, 2, CE), jnp.int32),
            pltpu.VMEM((KE * CE,), jnp.float32),
            [pltpu.VMEM((CE, dh), jnp.float32)] * _NB,
            [pltpu.VMEM((CE, dh), jnp.float32)] * _NB,
            pltpu.VMEM_SHARED((np_, dh), jnp.float32),
            [pltpu.SemaphoreType.DMA] * _NB,
            [pltpu.SemaphoreType.DMA] * _NB,
            [pltpu.SemaphoreType.DMA] * _NB,
        ],
        compiler_params=pltpu.CompilerParams(use_tc_tiling_on_sc=False),
    )
    def spmm_kernel(y_hbm, rc_hbm, ew_hbm, z_hbm, out_hbm,
                    rcv, ewv, gbufs, sbufs, acc, isems, gsems, ssems):
        c = lax.axis_index("c")
        s = lax.axis_index("s")
        pltpu.sync_copy(ew_hbm.at[s], ewv)
        pltpu.sync_copy(z_hbm, acc.at[pl.ds(s * slab, slab)])
        plsc.subcore_barrier()

        def idx_start(k, b):
            pltpu.async_copy(rc_hbm.at[s, k], rcv.at[b], isems[b])

        def idx_wait(k, b):
            pltpu.make_async_copy(rc_hbm.at[s, k], rcv.at[b], isems[b]).wait()

        def gather_start(k, b):
            pltpu.async_copy(y_hbm.at[c].at[rcv.at[b, 0]], gbufs[b], gsems[b])

        def gather_wait(k, b):
            pltpu.make_async_copy(y_hbm.at[c].at[rcv.at[b, 0]], gbufs[b], gsems[b]).wait()

        def scatter_start(k, b):
            pltpu.async_copy(sbufs[b], acc.at[rcv.at[b, 1]], ssems[b], add=True)

        def scatter_wait(k, b):
            pltpu.make_async_copy(sbufs[b], acc.at[rcv.at[b, 1]], ssems[b]).wait()

        def scale(k, b):
            gbuf, sbuf = gbufs[b], sbufs[b]

            def scale_group(g, carry2):
                v = ewv[pl.ds(k * CE + g * 16, 16)]
                for i2 in range(16):
                    sc = v[i2]
                    r = g * 16 + i2
                    for j in range(dh // 16):
                        sbuf[r, pl.ds(j * 16, 16)] = gbuf[r, pl.ds(j * 16, 16)] * sc
                return carry2

            lax.fori_loop(0, CE // 16, scale_group, 0)

        idx_start(0, 0)
        idx_start(1, 1)
        idx_start(2, 2)
        idx_wait(0, 0)
        gather_start(0, 0)
        idx_wait(1, 1)
        gather_start(1, 1)

        def round_body(r, carry):
            for b in range(_NB):
                k = r * _NB + b
                b2 = (b + 2) % _NB
                b3 = (b + 3) % _NB

                @pl.when(k >= 2)
                def _():
                    scatter_wait(k - 2, b3)

                @pl.when(k + 3 < KE)
                def _():
                    idx_start(k + 3, b3)

                @pl.when(k + 2 < KE)
                def _():
                    idx_wait(k + 2, b2)
                    gather_start(k + 2, b2)

                gather_wait(k, b)
                scale(k, b)
                scatter_start(k, b)
            return carry

        lax.fori_loop(0, KE // _NB, round_body, 0)

        scatter_wait(KE - 2, (KE - 2) % _NB)
        scatter_wait(KE - 1, (KE - 1) % _NB)

        plsc.subcore_barrier()
        pltpu.sync_copy(acc.at[pl.ds(s * slab, slab)], out_hbm.at[c, pl.ds(s * slab, slab)])

    return spmm_kernel(y2h, rcr, ewr, zslab)


def _dinv_pass(degp4):
    def body(dp_ref, o_ref):
        deg = dp_ref[0] + dp_ref[1] + 1.0
        o_ref[...] = jnp.where(deg > 0, lax.rsqrt(jnp.maximum(deg, 1e-12)), 0.0)

    return pl.pallas_call(
        body, out_shape=jax.ShapeDtypeStruct(degp4.shape[1:], jnp.float32)
    )(degp4)


_BLK = 1280


def _mm_scale_pass(x, W, dinvb):
    n, _ = x.shape
    d = W.shape[1]
    dh = d // 2
    g = n // _BLK

    def body(x_ref, w_ref, dv_ref, o_ref):
        xw = jnp.dot(x_ref[...], w_ref[...], preferred_element_type=jnp.float32)
        t = dv_ref[...] * xw
        o_ref[0] = t[:, :dh]
        o_ref[1] = t[:, dh:]

    return pl.pallas_call(
        body,
        grid=(g,),
        in_specs=[
            pl.BlockSpec((_BLK, x.shape[1]), lambda i: (i, 0)),
            pl.BlockSpec(W.shape, lambda i: (0, 0)),
            pl.BlockSpec((_BLK, d), lambda i: (i, 0)),
        ],
        out_specs=pl.BlockSpec((2, _BLK, dh), lambda i: (0, i, 0)),
        out_shape=jax.ShapeDtypeStruct((2, n, dh), jnp.float32),
    )(x, W, dinvb)


def _mid_pass(accp, y1, dinv1b, b1, W2, dinv2b):
    _, n, dh = y1.shape
    d = 2 * dh
    d2 = W2.shape[1]
    d2h = d2 // 2
    g = n // _BLK

    def body(a_ref, y_ref, d1_ref, b_ref, w_ref, d2_ref, o_ref):
        acc = jnp.concatenate([a_ref[0], a_ref[1]], axis=1)
        y = jnp.concatenate([y_ref[0], y_ref[1]], axis=1)
        t = d1_ref[...] * (acc + y) + b_ref[...]
        h = jnp.maximum(t, 0.0)
        hw = jnp.dot(h, w_ref[...], preferred_element_type=jnp.float32)
        t2 = d2_ref[...] * hw
        o_ref[0] = t2[:, :d2h]
        o_ref[1] = t2[:, d2h:]

    return pl.pallas_call(
        body,
        grid=(g,),
        in_specs=[
            pl.BlockSpec((2, _BLK, dh), lambda i: (0, i, 0)),
            pl.BlockSpec((2, _BLK, dh), lambda i: (0, i, 0)),
            pl.BlockSpec((_BLK, d), lambda i: (i, 0)),
            pl.BlockSpec(b1.shape, lambda i: (0, 0)),
            pl.BlockSpec(W2.shape, lambda i: (0, 0)),
            pl.BlockSpec((_BLK, d2), lambda i: (i, 0)),
        ],
        out_specs=pl.BlockSpec((2, _BLK, d2h), lambda i: (0, i, 0)),
        out_shape=jax.ShapeDtypeStruct((2, n, d2h), jnp.float32),
    )(accp, y1, dinv1b, b1, W2, dinv2b)


def _final_pass(accp, y2, dinv2b, b2):
    _, n, dh = y2.shape
    d = 2 * dh
    g = n // _BLK

    def body(a_ref, y_ref, dv_ref, b_ref, o_ref):
        acc = jnp.concatenate([a_ref[0], a_ref[1]], axis=1)
        y = jnp.concatenate([y_ref[0], y_ref[1]], axis=1)
        o_ref[...] = dv_ref[...] * (acc + y) + b_ref[...]

    return pl.pallas_call(
        body,
        grid=(g,),
        in_specs=[
            pl.BlockSpec((2, _BLK, dh), lambda i: (0, i, 0)),
            pl.BlockSpec((2, _BLK, dh), lambda i: (0, i, 0)),
            pl.BlockSpec((_BLK, d), lambda i: (i, 0)),
            pl.BlockSpec(b2.shape, lambda i: (0, 0)),
        ],
        out_specs=pl.BlockSpec((_BLK, d), lambda i: (i, 0)),
        out_shape=jax.ShapeDtypeStruct((n, d), jnp.float32),
    )(accp, y2, dinv2b, b2)


def kernel(x, edge_index, edge_attr, W1, b1, W2, b2):
    n, d_in = x.shape
    e = edge_index.shape[1]
    d_hid = W1.shape[1]
    d_out = W2.shape[1]
    np_ = ((n + _BLK - 1) // _BLK) * _BLK

    row = edge_index[0]
    col = edge_index[1]
    ew1 = edge_attr[:, 0]
    ew2 = edge_attr[:, 1]

    CE = 80
    KE = e // (NS * CE)
    rcr = jnp.stack([row.reshape(NS, KE, CE), col.reshape(NS, KE, CE)], axis=2)
    ew1r = ew1.reshape(NS, KE * CE)
    ew2r = ew2.reshape(NS, KE * CE)
    CD = 125
    KD = e // (NW * CD)
    colrd = col.reshape(NW, KD, CD)
    e1rd = ew1.reshape(NW, KD, CD)
    e2rd = ew2.reshape(NW, KD, CD)

    dh = d_hid // 2
    zdeg = jnp.zeros((np_ // NS,), jnp.float32)
    zacc = jnp.zeros((np_ // NS, dh), jnp.float32)
    x_p = jnp.pad(x, ((0, np_ - n), (0, 0)))

    degp = _degree_pass(colrd, e1rd, e2rd, zdeg, np_)
    dinvp = _dinv_pass(degp.reshape(NC, 2, np_ // 128, 128))
    dinv1b = jnp.broadcast_to(dinvp[0].reshape(np_, 1), (np_, d_hid))
    dinv2b = jnp.broadcast_to(dinvp[1].reshape(np_, 1), (np_, d_out))

    y1 = _mm_scale_pass(x_p, W1, dinv1b)
    acc1 = _spmm_pass(y1, rcr, ew1r, zacc, np_, dh, (KE, CE))
    y2 = _mid_pass(acc1, y1, dinv1b, b1.reshape(1, -1), W2, dinv2b)
    acc2 = _spmm_pass(y2, rcr, ew2r, zacc, np_, d_out // 2, (KE, CE))
    outp = _final_pass(acc2, y2, dinv2b, b2.reshape(1, -1))
    return outp[:n]
